# baseline (speedup 1.0000x reference)
import jax
import jax.numpy as jnp
from jax import lax
from jax.experimental import pallas as pl
from jax.experimental.pallas import tpu as pltpu
from jax.experimental.pallas import tpu_sc as plsc

_KN = 8192
_D = 32
_L = 16
_NS = 16
_CHUNK = _KN // _NS
_NG = _CHUNK // _L
_NC = _CHUNK // 128
_NITER = 1000
_A_START = 0.05
_A_END = 0.01
_THR = 0.5
_BIG = 3.0e38
_IBIG = 2**31 - 1


def _som_body(data_h, nodes_h, nhbr_h, i16_h, a16_h, out_h,
              chunk_v, xi_v, i_v, a_v, idx_v, col_v, am_v,
              shmin, shidx, red_min_v, red_idx_v, res_v, resi_v,
              sem1, sem2):
    sid = lax.axis_index("s")
    base = sid * _CHUNK
    iota = lax.iota(jnp.int32, _L)

    pltpu.sync_copy(i16_h, i_v)
    pltpu.sync_copy(a16_h, a_v)
    nd_cp = pltpu.async_copy(nodes_h.at[pl.ds(base * _D, _CHUNK * _D)], chunk_v, sem1)
    xi_cp = pltpu.async_copy(data_h.at[i_v], xi_v, sem2)
    nd_cp.wait()
    xi_cp.wait()

    xh = [xi_v[0, pl.ds(h * _L, _L)] for h in range(_D // _L)]
    xds = [jnp.full((_L,), xh[d // _L][d % _L], jnp.float32) for d in range(_D)]

    def group_step(g, carry):
        vmin, vidx = carry
        rows = iota + g * _L
        acc = jnp.zeros((_L,), jnp.float32)
        rowsd = rows * _D
        for d in range(_D):
            nv = plsc.load_gather(chunk_v, [rowsd + d])
            diff = nv - xds[d]
            acc = acc + diff * diff
        better = acc < vmin
        vmin = jnp.where(better, acc, vmin)
        vidx = jnp.where(better, rows + base, vidx)
        return vmin, vidx

    vmin0 = jnp.full((_L,), _BIG, jnp.float32)
    vidx0 = jnp.zeros((_L,), jnp.int32)
    vmin, vidx = lax.fori_loop(0, _NG, group_step, (vmin0, vidx0))

    lmin = jnp.min(vmin)
    lidx = jnp.min(jnp.where(vmin == lmin, vidx, _IBIG))

    res_v[...] = jnp.full((_L,), lmin, jnp.float32)
    resi_v[...] = jnp.full((_L,), lidx, jnp.int32)
    pltpu.sync_copy(res_v, shmin.at[sid])
    pltpu.sync_copy(resi_v, shidx.at[sid])
    plsc.subcore_barrier()
    pltpu.sync_copy(shmin, red_min_v)
    pltpu.sync_copy(shidx, red_idx_v)

    gmin = red_min_v[0, :]
    for w in range(1, _NS):
        gmin = jnp.minimum(gmin, red_min_v[w, :])
    near = jnp.full((_L,), _IBIG, jnp.int32)
    for w in range(_NS):
        cand = jnp.where(red_min_v[w, :] == gmin, red_idx_v[w, :], _IBIG)
        near = jnp.minimum(near, cand)

    for c in range(_NC):
        for g in range(128 // _L):
            rows = iota + (base + c * 128 + g * _L)
            idx_v[c, pl.ds(g * _L, _L)] = rows * _KN + near
    gathers = [pltpu.async_copy(nhbr_h.at[idx_v.at[c]], col_v.at[c], sem2)
               for c in range(_NC)]
    for cp in gathers:
        cp.wait()

    alpha_vec = a_v[...]
    zero = jnp.zeros((_L,), jnp.float32)
    for c in range(_NC):
        for g in range(128 // _L):
            cv = col_v[c, pl.ds(g * _L, _L)]
            am_v[pl.ds(c * 128 + g * _L, _L)] = jnp.where(cv <= _THR, alpha_vec, zero)

    def upd(g, _):
        amg = am_v[pl.ds(g * _L, _L)]
        for l in range(_L):
            off = (g * _L + l) * _D
            a = amg[l]
            for h in range(_D // _L):
                row = chunk_v[pl.ds(off + h * _L, _L)]
                chunk_v[pl.ds(off + h * _L, _L)] = row + (xh[h] - row) * a
        return 0
    lax.fori_loop(0, _NG, upd, 0)

    pltpu.sync_copy(chunk_v, out_h.at[pl.ds(base * _D, _CHUNK * _D)])


_som_call = pl.kernel(
    _som_body,
    out_type=jax.ShapeDtypeStruct((_KN * _D,), jnp.float32),
    mesh=plsc.VectorSubcoreMesh(core_axis_name="c", subcore_axis_name="s",
                                num_cores=1, num_subcores=_NS),
    compiler_params=pltpu.CompilerParams(needs_layout_passes=False,
                                         use_tc_tiling_on_sc=False),
    scratch_types=[
        pltpu.VMEM((_CHUNK * _D,), jnp.float32),
        pltpu.VMEM((_L, _D), jnp.float32),
        pltpu.VMEM((_L,), jnp.int32),
        pltpu.VMEM((_L,), jnp.float32),
        pltpu.VMEM((_NC, 128), jnp.int32),
        pltpu.VMEM((_NC, 128), jnp.float32),
        pltpu.VMEM((_CHUNK,), jnp.float32),
        pltpu.VMEM_SHARED((_NS, _L), jnp.float32),
        pltpu.VMEM_SHARED((_NS, _L), jnp.int32),
        pltpu.VMEM((_NS, _L), jnp.float32),
        pltpu.VMEM((_NS, _L), jnp.int32),
        pltpu.VMEM((_L,), jnp.float32),
        pltpu.VMEM((_L,), jnp.int32),
        pltpu.SemaphoreType.DMA,
        pltpu.SemaphoreType.DMA,
    ],
)


def kernel(data, nodes, nhbrdist, rand_indices, k):
    i = rand_indices[k]
    alpha = jnp.float32(_A_START) - jnp.float32(_A_START - _A_END) * (k / _NITER)
    i16 = jnp.full((_L,), i, jnp.int32)
    a16 = jnp.full((_L,), alpha, jnp.float32)
    nhbr_flat = nhbrdist.reshape(-1)
    nodes_flat = nodes.reshape(-1)
    out = _som_call(data, nodes_flat, nhbr_flat, i16, a16)
    return out.reshape(_KN, _D)


# speedup vs baseline: 9.2700x; 9.2700x over previous
import jax
import jax.numpy as jnp
from jax import lax
from jax.experimental import pallas as pl
from jax.experimental.pallas import tpu as pltpu
from jax.experimental.pallas import tpu_sc as plsc

_KN = 8192
_D = 32
_L = 16
_NS = 16
_CHUNK = _KN // _NS
_NG = _CHUNK // _L
_NC = _CHUNK // 128
_RSTRIDE = 64 * 1024
_NITER = 1000
_A_START = 0.05
_A_END = 0.01
_THR = 0.5
_BIG = 3.0e38
_IBIG = 2**31 - 1


def _som_body(nodes_h, nhbr_h, xi_h, a16_h, out_h,
              chunk_v, xi_v, a_v, idx_v, col_v, am_v,
              shmin, shidx, red_min_v, red_idx_v, res_v, resi_v,
              sem1, sem2):
    sid = lax.axis_index("s")
    base = sid * _CHUNK
    iota = lax.iota(jnp.int32, _L)

    pltpu.sync_copy(xi_h, xi_v)
    pltpu.sync_copy(a16_h, a_v)
    cps = []
    for r in range(_D // 8):
        for c in range(_NC):
            cps.append(pltpu.async_copy(
                nodes_h.at[pl.ds(r * _RSTRIDE + (sid * _NC + c) * 1024, 1024)],
                chunk_v.at[pl.ds(r * _NC * 1024 + c * 1024, 1024)], sem1))
    for cp in cps:
        cp.wait()

    xh = [xi_v[pl.ds(h * _L, _L)] for h in range(_D // _L)]
    xds = [jnp.full((_L,), xh[d // _L][d % _L], jnp.float32) for d in range(_D)]

    def group_step(m, carry):
        vmin, vidx = carry
        dyn = (m >> 3) * 1024 + (m & 7) * _L
        acc = jnp.zeros((_L,), jnp.float32)
        for d in range(_D):
            nv = chunk_v[pl.ds(dyn + (d >> 3) * 4096 + (d & 7) * 128, _L)]
            diff = nv - xds[d]
            acc = acc + diff * diff
        better = acc < vmin
        vmin = jnp.where(better, acc, vmin)
        vidx = jnp.where(better, base + m * _L + iota, vidx)
        return vmin, vidx

    vmin0 = jnp.full((_L,), _BIG, jnp.float32)
    vidx0 = jnp.zeros((_L,), jnp.int32)
    vmin, vidx = lax.fori_loop(0, _NG, group_step, (vmin0, vidx0))

    lmin = jnp.min(vmin)
    lidx = jnp.min(jnp.where(vmin == lmin, vidx, _IBIG))

    res_v[...] = jnp.full((_L,), lmin, jnp.float32)
    resi_v[...] = jnp.full((_L,), lidx, jnp.int32)
    pltpu.sync_copy(res_v, shmin.at[sid])
    pltpu.sync_copy(resi_v, shidx.at[sid])
    plsc.subcore_barrier()
    pltpu.sync_copy(shmin, red_min_v)
    pltpu.sync_copy(shidx, red_idx_v)

    gmin = red_min_v[0, :]
    for w in range(1, _NS):
        gmin = jnp.minimum(gmin, red_min_v[w, :])
    near = jnp.full((_L,), _IBIG, jnp.int32)
    for w in range(_NS):
        cand = jnp.where(red_min_v[w, :] == gmin, red_idx_v[w, :], _IBIG)
        near = jnp.minimum(near, cand)

    ncol = ((near >> 7) << 10) + (near & 127)
    for c in range(_NC):
        for g in range(128 // _L):
            n = base + c * 128 + g * _L + iota
            flat = ((n >> 3) << 16) + ((n & 7) << 7) + ncol
            idx_v[c, pl.ds(g * _L, _L)] = flat
    gathers = [pltpu.async_copy(nhbr_h.at[idx_v.at[c]], col_v.at[c], sem2)
               for c in range(_NC)]
    for cp in gathers:
        cp.wait()

    alpha_vec = a_v[...]
    zero = jnp.zeros((_L,), jnp.float32)
    for c in range(_NC):
        for g in range(128 // _L):
            cv = col_v[c, pl.ds(g * _L, _L)]
            am_v[pl.ds(c * 128 + g * _L, _L)] = jnp.where(cv <= _THR, alpha_vec, zero)

    def upd(m, _):
        dyn = (m >> 3) * 1024 + (m & 7) * _L
        amg = am_v[pl.ds(m * _L, _L)]
        for d in range(_D):
            off = dyn + (d >> 3) * 4096 + (d & 7) * 128
            v = chunk_v[pl.ds(off, _L)]
            chunk_v[pl.ds(off, _L)] = v + (xds[d] - v) * amg
        return 0
    lax.fori_loop(0, _NG, upd, 0)

    outs = []
    for r in range(_D // 8):
        for c in range(_NC):
            outs.append(pltpu.async_copy(
                chunk_v.at[pl.ds(r * _NC * 1024 + c * 1024, 1024)],
                out_h.at[pl.ds(r * _RSTRIDE + (sid * _NC + c) * 1024, 1024)],
                sem1))
    for cp in outs:
        cp.wait()


_som_call = pl.kernel(
    _som_body,
    out_type=jax.ShapeDtypeStruct((_KN * _D,), jnp.float32),
    mesh=plsc.VectorSubcoreMesh(core_axis_name="c", subcore_axis_name="s",
                                num_cores=1, num_subcores=_NS),
    compiler_params=pltpu.CompilerParams(needs_layout_passes=False,
                                         use_tc_tiling_on_sc=False),
    scratch_types=[
        pltpu.VMEM((_CHUNK * _D,), jnp.float32),
        pltpu.VMEM((_D,), jnp.float32),
        pltpu.VMEM((_L,), jnp.float32),
        pltpu.VMEM((_NC, 128), jnp.int32),
        pltpu.VMEM((_NC, 128), jnp.float32),
        pltpu.VMEM((_CHUNK,), jnp.float32),
        pltpu.VMEM_SHARED((_NS, _L), jnp.float32),
        pltpu.VMEM_SHARED((_NS, _L), jnp.int32),
        pltpu.VMEM((_NS, _L), jnp.float32),
        pltpu.VMEM((_NS, _L), jnp.int32),
        pltpu.VMEM((_L,), jnp.float32),
        pltpu.VMEM((_L,), jnp.int32),
        pltpu.SemaphoreType.DMA,
        pltpu.SemaphoreType.DMA,
    ],
)


def kernel(data, nodes, nhbrdist, rand_indices, k):
    i = rand_indices[k]
    alpha = jnp.float32(_A_START) - jnp.float32(_A_START - _A_END) * (k / _NITER)
    xi = data[i]
    a16 = jnp.full((_L,), alpha, jnp.float32)
    nhbr_lin = nhbrdist.reshape(1024, 8, 64, 128).transpose(0, 2, 1, 3).reshape(-1)
    nodes_lin = nodes.T.reshape(4, 8, 64, 128).transpose(0, 2, 1, 3).reshape(-1)
    out_lin = _som_call(nodes_lin, nhbr_lin, xi, a16)
    out_t = out_lin.reshape(4, 64, 8, 128).transpose(0, 2, 1, 3).reshape(_D, _KN)
    return out_t.T


# speedup vs baseline: 9.3492x; 1.0085x over previous
import jax
import jax.numpy as jnp
from jax import lax
from jax.experimental import pallas as pl
from jax.experimental.pallas import tpu as pltpu
from jax.experimental.pallas import tpu_sc as plsc

_KN = 8192
_D = 32
_L = 16
_NS = 16
_CHUNK = _KN // _NS
_NG = _CHUNK // _L
_NC = _CHUNK // 128
_RSTRIDE = 64 * 1024
_NITER = 1000
_A_START = 0.05
_A_END = 0.01
_THR = 0.5
_BIG = 3.0e38
_IBIG = 2**31 - 1


def _som_body(nodes_h, nhbr_h, xi_h, a16_h, out_h,
              chunk_v, xi_v, a_v, idx_v, col_v, am_v,
              shmin, shidx, red_min_v, red_idx_v, res_v, resi_v,
              sem1, sem2):
    sid = lax.axis_index("s")
    base = sid * _CHUNK
    iota = lax.iota(jnp.int32, _L)

    pltpu.sync_copy(xi_h, xi_v)
    pltpu.sync_copy(a16_h, a_v)

    def stage_in(t, _):
        r = t >> 2
        c = t & 3
        pltpu.async_copy(
            nodes_h.at[pl.ds(r * _RSTRIDE + (sid * _NC + c) * 1024, 1024)],
            chunk_v.at[pl.ds(t * 1024, 1024)], sem1)
        return 0
    lax.fori_loop(0, 16, stage_in, 0)
    pltpu.make_async_copy(nodes_h.at[pl.ds(0, _CHUNK * _D)], chunk_v, sem1).wait()

    xh = [xi_v[pl.ds(h * _L, _L)] for h in range(_D // _L)]
    xds = [jnp.full((_L,), xh[d // _L][d % _L], jnp.float32) for d in range(_D)]

    def group_step(m, carry):
        vmin, vidx = carry
        dyn = (m >> 3) * 1024 + (m & 7) * _L
        acc = jnp.zeros((_L,), jnp.float32)
        for d in range(_D):
            nv = chunk_v[pl.ds(dyn + (d >> 3) * 4096 + (d & 7) * 128, _L)]
            diff = nv - xds[d]
            acc = acc + diff * diff
        better = acc < vmin
        vmin = jnp.where(better, acc, vmin)
        vidx = jnp.where(better, base + m * _L + iota, vidx)
        return vmin, vidx

    vmin0 = jnp.full((_L,), _BIG, jnp.float32)
    vidx0 = jnp.zeros((_L,), jnp.int32)
    vmin, vidx = lax.fori_loop(0, _NG, group_step, (vmin0, vidx0))

    lmin = jnp.min(vmin)
    lidx = jnp.min(jnp.where(vmin == lmin, vidx, _IBIG))

    res_v[...] = jnp.full((_L,), lmin, jnp.float32)
    resi_v[...] = jnp.full((_L,), lidx, jnp.int32)
    pltpu.sync_copy(res_v, shmin.at[pl.ds(sid * _L, _L)])
    pltpu.sync_copy(resi_v, shidx.at[pl.ds(sid * _L, _L)])
    plsc.subcore_barrier()
    pltpu.sync_copy(shmin, red_min_v)
    pltpu.sync_copy(shidx, red_idx_v)

    def minstep(w, carry):
        return jnp.minimum(carry, red_min_v[pl.ds(w * _L, _L)])
    gmin = lax.fori_loop(1, _NS, minstep, red_min_v[pl.ds(0, _L)])

    def idxstep(w, carry):
        rm = red_min_v[pl.ds(w * _L, _L)]
        ri = red_idx_v[pl.ds(w * _L, _L)]
        return jnp.minimum(carry, jnp.where(rm == gmin, ri, _IBIG))
    near = lax.fori_loop(0, _NS, idxstep, jnp.full((_L,), _IBIG, jnp.int32))

    ncol = ((near >> 7) << 10) + (near & 127)

    def mkidx(m, _):
        n = base + m * _L + iota
        idx_v[m >> 3, pl.ds((m & 7) * _L, _L)] = ((n >> 3) << 16) + ((n & 7) << 7) + ncol
        return 0
    lax.fori_loop(0, _NG, mkidx, 0)

    gathers = [pltpu.async_copy(nhbr_h.at[idx_v.at[c]], col_v.at[c], sem2)
               for c in range(_NC)]
    for cp in gathers:
        cp.wait()

    alpha_vec = a_v[...]

    def mkam(m, _):
        cv = col_v[m >> 3, pl.ds((m & 7) * _L, _L)]
        am_v[pl.ds(m * _L, _L)] = jnp.where(cv <= _THR, alpha_vec, 0.0)
        return 0
    lax.fori_loop(0, _NG, mkam, 0)

    def upd(m, _):
        dyn = (m >> 3) * 1024 + (m & 7) * _L
        amg = am_v[pl.ds(m * _L, _L)]
        for d in range(_D):
            off = dyn + (d >> 3) * 4096 + (d & 7) * 128
            v = chunk_v[pl.ds(off, _L)]
            chunk_v[pl.ds(off, _L)] = v + (xds[d] - v) * amg
        return 0
    lax.fori_loop(0, _NG, upd, 0)

    def stage_out(t, _):
        r = t >> 2
        c = t & 3
        pltpu.async_copy(
            chunk_v.at[pl.ds(t * 1024, 1024)],
            out_h.at[pl.ds(r * _RSTRIDE + (sid * _NC + c) * 1024, 1024)], sem1)
        return 0
    lax.fori_loop(0, 16, stage_out, 0)
    pltpu.make_async_copy(chunk_v, out_h.at[pl.ds(0, _CHUNK * _D)], sem1).wait()


_som_call = pl.kernel(
    _som_body,
    out_type=jax.ShapeDtypeStruct((_KN * _D,), jnp.float32),
    mesh=plsc.VectorSubcoreMesh(core_axis_name="c", subcore_axis_name="s",
                                num_cores=1, num_subcores=_NS),
    compiler_params=pltpu.CompilerParams(needs_layout_passes=False,
                                         use_tc_tiling_on_sc=False),
    scratch_types=[
        pltpu.VMEM((_CHUNK * _D,), jnp.float32),
        pltpu.VMEM((_D,), jnp.float32),
        pltpu.VMEM((_L,), jnp.float32),
        pltpu.VMEM((_NC, 128), jnp.int32),
        pltpu.VMEM((_NC, 128), jnp.float32),
        pltpu.VMEM((_CHUNK,), jnp.float32),
        pltpu.VMEM_SHARED((_NS * _L,), jnp.float32),
        pltpu.VMEM_SHARED((_NS * _L,), jnp.int32),
        pltpu.VMEM((_NS * _L,), jnp.float32),
        pltpu.VMEM((_NS * _L,), jnp.int32),
        pltpu.VMEM((_L,), jnp.float32),
        pltpu.VMEM((_L,), jnp.int32),
        pltpu.SemaphoreType.DMA,
        pltpu.SemaphoreType.DMA,
    ],
)


def kernel(data, nodes, nhbrdist, rand_indices, k):
    i = rand_indices[k]
    alpha = jnp.float32(_A_START) - jnp.float32(_A_START - _A_END) * (k / _NITER)
    xi = data[i]
    a16 = jnp.full((_L,), alpha, jnp.float32)
    nhbr_lin = nhbrdist.reshape(1024, 8, 64, 128).transpose(0, 2, 1, 3).reshape(-1)
    nodes_lin = nodes.T.reshape(4, 8, 64, 128).transpose(0, 2, 1, 3).reshape(-1)
    out_lin = _som_call(nodes_lin, nhbr_lin, xi, a16)
    out_t = out_lin.reshape(4, 64, 8, 128).transpose(0, 2, 1, 3).reshape(_D, _KN)
    return out_t.T


# speedup vs baseline: 25.8089x; 2.7606x over previous
import jax
import jax.numpy as jnp
from jax import lax
from jax.experimental import pallas as pl
from jax.experimental.pallas import tpu as pltpu

_KN = 8192
_D = 32
_N = 100000
_NITER = 1000
_A_START = 0.05
_A_END = 0.01
_THR = 0.5
_IBIG = 2**31 - 1


def _tc_body(ridx_s, k_s, dataT_h, nodesT_v, nhbr_h, out_v,
             xiblk_v, colblk_v, sem1, sem2):
    k = k_s[0]
    i = ridx_s[k]
    alpha = jnp.float32(_A_START) - jnp.float32(_A_START - _A_END) * (
        k.astype(jnp.float32) / _NITER)

    ib = i // 128
    cp1 = pltpu.make_async_copy(
        dataT_h.at[:, pl.ds(ib * 128, 128)], xiblk_v, sem1)
    cp1.start()
    cp1.wait()
    j = i - ib * 128
    lane = lax.broadcasted_iota(jnp.int32, (1, 128), 1)
    xcol = jnp.sum(jnp.where(lane == j, xiblk_v[...], 0.0),
                   axis=1, keepdims=True)

    nt = nodesT_v[...]
    diff = nt - xcol
    dist2 = jnp.sum(diff * diff, axis=0, keepdims=True)
    m = jnp.min(dist2)
    nio = lax.broadcasted_iota(jnp.int32, (1, _KN), 1)
    nearest = jnp.min(jnp.where(dist2 == m, nio, _IBIG))

    cb = nearest // 128
    cp2 = pltpu.make_async_copy(
        nhbr_h.at[:, pl.ds(cb * 128, 128)], colblk_v, sem2)
    cp2.start()
    cp2.wait()
    jc = nearest - cb * 128
    onehot2 = (lane == jc).astype(jnp.float32)
    col_row = lax.dot_general(onehot2, colblk_v[...],
                              (((1,), (1,)), ((), ())),
                              precision=lax.Precision.HIGHEST,
                              preferred_element_type=jnp.float32)
    am = jnp.where(col_row <= _THR, alpha, 0.0)
    out_v[...] = nt + (xcol - nt) * am


_tc_call = pl.pallas_call(
    _tc_body,
    out_shape=jax.ShapeDtypeStruct((_D, _KN), jnp.float32),
    in_specs=[
        pl.BlockSpec(memory_space=pltpu.SMEM),
        pl.BlockSpec(memory_space=pltpu.SMEM),
        pl.BlockSpec(memory_space=pltpu.HBM),
        pl.BlockSpec(memory_space=pltpu.VMEM),
        pl.BlockSpec(memory_space=pltpu.HBM),
    ],
    out_specs=pl.BlockSpec(memory_space=pltpu.VMEM),
    scratch_shapes=[
        pltpu.VMEM((_D, 128), jnp.float32),
        pltpu.VMEM((_KN, 128), jnp.float32),
        pltpu.SemaphoreType.DMA,
        pltpu.SemaphoreType.DMA,
    ],
    compiler_params=pltpu.CompilerParams(
        dimension_semantics=(), vmem_limit_bytes=100 * 1024 * 1024),
)


def kernel(data, nodes, nhbrdist, rand_indices, k):
    karr = jnp.reshape(k, (1,)).astype(jnp.int32)
    out_t = _tc_call(rand_indices, karr, data.T, nodes.T, nhbrdist)
    return out_t.T


# speedup vs baseline: 36.8485x; 1.4277x over previous
import jax
import jax.numpy as jnp
from jax import lax
from jax.experimental import pallas as pl
from jax.experimental.pallas import tpu as pltpu

_KN = 8192
_D = 32
_N = 100000
_NITER = 1000
_A_START = 0.05
_A_END = 0.01
_THR = 0.5
_IBIG = 2**31 - 1
_S = 4
_RB = _KN // _S


def _tc_body(ridx_s, k_s, dataT_h, nodesT_v, nhbr_h, out_v,
             xiblk_v, colblk_v, sem1, csems):
    k = k_s[0]
    i = ridx_s[k]
    alpha = jnp.float32(_A_START) - jnp.float32(_A_START - _A_END) * (
        k.astype(jnp.float32) / _NITER)

    ib = i // 128
    cp1 = pltpu.make_async_copy(
        dataT_h.at[:, pl.ds(ib * 128, 128)], xiblk_v, sem1)
    cp1.start()
    cp1.wait()
    j = i - ib * 128
    lane = lax.broadcasted_iota(jnp.int32, (1, 128), 1)
    xcol = jnp.sum(jnp.where(lane == j, xiblk_v[...], 0.0),
                   axis=1, keepdims=True)

    nt = nodesT_v[...]
    diff = nt - xcol
    dist2 = jnp.sum(diff * diff, axis=0, keepdims=True)
    m = jnp.min(dist2)
    nio = lax.broadcasted_iota(jnp.int32, (1, _KN), 1)
    nearest = jnp.min(jnp.where(dist2 == m, nio, _IBIG))

    cb = nearest // 128
    cps = []
    for c in range(_S):
        cp = pltpu.make_async_copy(
            nhbr_h.at[pl.ds(c * _RB, _RB), pl.ds(cb * 128, 128)],
            colblk_v.at[pl.ds(c * _RB, _RB)], csems.at[c])
        cp.start()
        cps.append(cp)
    jc = nearest - cb * 128
    onehot2 = (lane == jc).astype(jnp.float32)
    for c in range(_S):
        cps[c].wait()
        mchunk = (colblk_v[pl.ds(c * _RB, _RB), :] <= _THR).astype(jnp.float32)
        colm = lax.dot_general(onehot2, mchunk, (((1,), (1,)), ((), ())),
                               preferred_element_type=jnp.float32)
        am = colm * alpha
        ntc = nt[:, c * _RB:(c + 1) * _RB]
        out_v[:, pl.ds(c * _RB, _RB)] = ntc + (xcol - ntc) * am


_tc_call = pl.pallas_call(
    _tc_body,
    out_shape=jax.ShapeDtypeStruct((_D, _KN), jnp.float32),
    in_specs=[
        pl.BlockSpec(memory_space=pltpu.SMEM),
        pl.BlockSpec(memory_space=pltpu.SMEM),
        pl.BlockSpec(memory_space=pltpu.HBM),
        pl.BlockSpec(memory_space=pltpu.VMEM),
        pl.BlockSpec(memory_space=pltpu.HBM),
    ],
    out_specs=pl.BlockSpec(memory_space=pltpu.VMEM),
    scratch_shapes=[
        pltpu.VMEM((_D, 128), jnp.float32),
        pltpu.VMEM((_KN, 128), jnp.float32),
        pltpu.SemaphoreType.DMA,
        pltpu.SemaphoreType.DMA((_S,)),
    ],
    compiler_params=pltpu.CompilerParams(
        dimension_semantics=(), vmem_limit_bytes=100 * 1024 * 1024),
)


def kernel(data, nodes, nhbrdist, rand_indices, k):
    karr = jnp.reshape(k, (1,)).astype(jnp.int32)
    out_t = _tc_call(rand_indices, karr, data.T, nodes.T, nhbrdist)
    return out_t.T
